# baseline (speedup 1.0000x reference)
import functools

import jax
import jax.numpy as jnp
from jax import lax
from jax.experimental import pallas as pl
from jax.experimental.pallas import tpu as pltpu
from jax.experimental.pallas import tpu_sc as plsc

INPUT_DIM = 100000
LAYER1_DIM = 64
HID_DIM = 128
BATCH = 4096
HIST = 50

NW = 32
BPW = BATCH // NW
NCH = HIST
NBUF = 7
NSPLIT = 4

COLS_BLK = 25088


def _proj_body(w1t_ref, w2t_ref, out_ref):
    out_ref[...] = lax.dot_general(
        w1t_ref[...], w2t_ref[...],
        dimension_numbers=(((0,), (0,)), ((), ())),
        preferred_element_type=jnp.float32,
    )


def _project(emb1_t, emb2_t):
    return pl.pallas_call(
        _proj_body,
        grid=(pl.cdiv(INPUT_DIM, COLS_BLK),),
        in_specs=[
            pl.BlockSpec((LAYER1_DIM, COLS_BLK), lambda i: (0, i)),
            pl.BlockSpec((LAYER1_DIM, HID_DIM), lambda i: (0, 0)),
        ],
        out_specs=pl.BlockSpec((COLS_BLK, HID_DIM), lambda i: (i, 0)),
        out_shape=jax.ShapeDtypeStruct((INPUT_DIM, HID_DIM), jnp.float32),
    )(emb1_t, emb2_t)


@functools.lru_cache(maxsize=1)
def _build_gather():
    mesh = plsc.VectorSubcoreMesh(core_axis_name="c", subcore_axis_name="s")

    @functools.partial(
        pl.kernel,
        mesh=mesh,
        out_type=jax.ShapeDtypeStruct((HIST * NW, BPW, HID_DIM), jnp.float32),
        scratch_types=[
            pltpu.VMEM((NCH, BPW), jnp.int32),
            *[pltpu.VMEM((BPW, HID_DIM), jnp.float32) for _ in range(NBUF)],
            *[pltpu.SemaphoreType.DMA for _ in range(2 * NBUF)],
        ],
    )
    def _gather(proj_hbm, idx_hbm, out_hbm, idx_v, *scratch):
        bufs = scratch[:NBUF]
        gsems = scratch[NBUF:2 * NBUF]
        wsems = scratch[2 * NBUF:]
        wid = lax.axis_index("s") * 2 + lax.axis_index("c")
        pltpu.sync_copy(idx_hbm.at[:, pl.ds(wid * BPW, BPW)], idx_v)
        sub = BPW // NSPLIT

        def start_gather(j, b):
            for h in range(NSPLIT):
                pltpu.async_copy(
                    proj_hbm.at[idx_v.at[j, pl.ds(h * sub, sub)]],
                    bufs[b].at[pl.ds(h * sub, sub)],
                    gsems[b],
                )

        for b in range(NBUF):
            start_gather(b, b)

        def wait_writes(j, b):
            for h in range(NSPLIT):
                pltpu.make_async_copy(
                    bufs[b].at[pl.ds(h * sub, sub)],
                    out_hbm.at[j * NW + wid].at[pl.ds(h * sub, sub)],
                    wsems[b],
                ).wait()

        def body(t, carry):
            i = t * NBUF
            for b in range(NBUF):
                j = i + b
                dst = out_hbm.at[j * NW + wid]
                for h in range(NSPLIT):
                    pltpu.make_async_copy(
                        proj_hbm.at[idx_v.at[j, pl.ds(h * sub, sub)]],
                        bufs[b].at[pl.ds(h * sub, sub)],
                        gsems[b],
                    ).wait()
                    pltpu.async_copy(
                        bufs[b].at[pl.ds(h * sub, sub)],
                        dst.at[pl.ds(h * sub, sub)],
                        wsems[b],
                    )

                @pl.when(j + NBUF < NCH)
                def _():
                    wait_writes(j, b)
                    start_gather(j + NBUF, b)

            return carry

        lax.fori_loop(0, NCH // NBUF, body, jnp.int32(0))

        for j in range((NCH // NBUF) * NBUF, NCH):
            b = j % NBUF
            dst = out_hbm.at[j * NW + wid]
            for h in range(NSPLIT):
                pltpu.make_async_copy(
                    proj_hbm.at[idx_v.at[j, pl.ds(h * sub, sub)]],
                    bufs[b].at[pl.ds(h * sub, sub)],
                    gsems[b],
                ).wait()
                pltpu.async_copy(
                    bufs[b].at[pl.ds(h * sub, sub)],
                    dst.at[pl.ds(h * sub, sub)],
                    wsems[b],
                )

        for j in range(NCH - NBUF, NCH):
            wait_writes(j, j % NBUF)

    return _gather


def kernel(src, emb1_weight, emb2_weight):
    proj = _project(emb1_weight.T, emb2_weight.T)
    out = _build_gather()(proj, src.T)
    return out.reshape(HIST, BATCH, HID_DIM).transpose(1, 0, 2)


# speedup vs baseline: 1.0043x; 1.0043x over previous
import functools

import jax
import jax.numpy as jnp
from jax import lax
from jax.experimental import pallas as pl
from jax.experimental.pallas import tpu as pltpu
from jax.experimental.pallas import tpu_sc as plsc

INPUT_DIM = 100000
LAYER1_DIM = 64
HID_DIM = 128
BATCH = 4096
HIST = 50

NW = 32
BPW = BATCH // NW
NCH = HIST
NBUF = 5
NSPLIT = 4

COLS_BLK = 25088


def _proj_body(w1t_ref, w2t_ref, out_ref):
    out_ref[...] = lax.dot_general(
        w1t_ref[...], w2t_ref[...],
        dimension_numbers=(((0,), (0,)), ((), ())),
        preferred_element_type=jnp.float32,
    )


def _project(emb1_t, emb2_t):
    return pl.pallas_call(
        _proj_body,
        grid=(pl.cdiv(INPUT_DIM, COLS_BLK),),
        in_specs=[
            pl.BlockSpec((LAYER1_DIM, COLS_BLK), lambda i: (0, i)),
            pl.BlockSpec((LAYER1_DIM, HID_DIM), lambda i: (0, 0)),
        ],
        out_specs=pl.BlockSpec((COLS_BLK, HID_DIM), lambda i: (i, 0)),
        out_shape=jax.ShapeDtypeStruct((INPUT_DIM, HID_DIM), jnp.float32),
    )(emb1_t, emb2_t)


@functools.lru_cache(maxsize=1)
def _build_gather():
    mesh = plsc.VectorSubcoreMesh(core_axis_name="c", subcore_axis_name="s")

    @functools.partial(
        pl.kernel,
        mesh=mesh,
        out_type=jax.ShapeDtypeStruct((HIST * NW, BPW, HID_DIM), jnp.float32),
        scratch_types=[
            pltpu.VMEM((NCH, BPW), jnp.int32),
            *[pltpu.VMEM((BPW, HID_DIM), jnp.float32) for _ in range(NBUF)],
            *[pltpu.SemaphoreType.DMA for _ in range(2 * NBUF)],
        ],
    )
    def _gather(proj_hbm, idx_hbm, out_hbm, idx_v, *scratch):
        bufs = scratch[:NBUF]
        gsems = scratch[NBUF:2 * NBUF]
        wsems = scratch[2 * NBUF:]
        wid = lax.axis_index("s") * 2 + lax.axis_index("c")
        pltpu.sync_copy(idx_hbm.at[:, pl.ds(wid * BPW, BPW)], idx_v)
        sub = BPW // NSPLIT

        def start_gather(j, b):
            for h in range(NSPLIT):
                pltpu.async_copy(
                    proj_hbm.at[idx_v.at[j, pl.ds(h * sub, sub)]],
                    bufs[b].at[pl.ds(h * sub, sub)],
                    gsems[b],
                )

        for b in range(NBUF):
            start_gather(b, b)

        def wait_writes(j, b):
            for h in range(NSPLIT):
                pltpu.make_async_copy(
                    bufs[b].at[pl.ds(h * sub, sub)],
                    out_hbm.at[j * NW + wid].at[pl.ds(h * sub, sub)],
                    wsems[b],
                ).wait()

        def body(t, carry):
            i = t * NBUF
            for b in range(NBUF):
                j = i + b
                dst = out_hbm.at[j * NW + wid]
                for h in range(NSPLIT):
                    pltpu.make_async_copy(
                        proj_hbm.at[idx_v.at[j, pl.ds(h * sub, sub)]],
                        bufs[b].at[pl.ds(h * sub, sub)],
                        gsems[b],
                    ).wait()
                    pltpu.async_copy(
                        bufs[b].at[pl.ds(h * sub, sub)],
                        dst.at[pl.ds(h * sub, sub)],
                        wsems[b],
                    )

                @pl.when(j + NBUF < NCH)
                def _():
                    wait_writes(j, b)
                    start_gather(j + NBUF, b)

            return carry

        lax.fori_loop(0, NCH // NBUF, body, jnp.int32(0))

        for b in range(NBUF):
            wait_writes(NCH - NBUF + b, b)

    return _gather


def kernel(src, emb1_weight, emb2_weight):
    proj = _project(emb1_weight.T, emb2_weight.T)
    out = _build_gather()(proj, src.T)
    return out.reshape(HIST, BATCH, HID_DIM).transpose(1, 0, 2)
